# Initial kernel scaffold; baseline (speedup 1.0000x reference)
import jax
import jax.numpy as jnp
from jax.experimental import pallas as pl


def kernel(x, embeddings):
    raise NotImplementedError("write your pallas kernel here")


# speedup vs baseline: 1.5321x; 1.5321x over previous
import functools

import jax
import jax.numpy as jnp
from jax import lax
from jax.experimental import pallas as pl
from jax.experimental.pallas import tpu as pltpu
from jax.experimental.pallas import tpu_sc as plsc

_NUM_EMB = 512
_DIM = 32
_BETA = 0.25
_TOK = 64 * 1024
_BLK = 2048
_NW = 32
_PER_W = _TOK // _NW
_JCH = 128
_NJ = _PER_W // _JCH


def _vq_tc_body(x_ref, emb_ref, idx_ref, loss_ref):
    xb = x_ref[...]
    emb = emb_ref[...]
    sim = lax.dot_general(xb, emb, (((1,), (0,)), ((), ())),
                          preferred_element_type=jnp.float32)
    x2 = jnp.sum(xb * xb, axis=1, keepdims=True)
    e2 = jnp.sum(emb * emb, axis=0, keepdims=True)
    dist = x2 + e2 - 2.0 * sim
    m = jnp.min(dist, axis=1, keepdims=True)
    iota = lax.broadcasted_iota(jnp.int32, dist.shape, 1)
    idx = jnp.min(jnp.where(dist == m, iota, _NUM_EMB), axis=1, keepdims=True)
    idx_ref[...] = idx

    @pl.when(pl.program_id(0) == 0)
    def _():
        loss_ref[0, 0] = 0.0

    loss_ref[0, 0] += jnp.sum(m)


def _tc_stage(x_flat, emb):
    return pl.pallas_call(
        _vq_tc_body,
        grid=(_TOK // _BLK,),
        in_specs=[
            pl.BlockSpec((_BLK, _DIM), lambda i: (i, 0)),
            pl.BlockSpec((_DIM, _NUM_EMB), lambda i: (0, 0)),
        ],
        out_specs=[
            pl.BlockSpec((_BLK, 1), lambda i: (i, 0)),
            pl.BlockSpec((1, 1), lambda i: (0, 0), memory_space=pltpu.SMEM),
        ],
        out_shape=[
            jax.ShapeDtypeStruct((_TOK, 1), jnp.int32),
            jax.ShapeDtypeStruct((1, 1), jnp.float32),
        ],
        compiler_params=pltpu.CompilerParams(
            dimension_semantics=("arbitrary",)),
    )(x_flat, emb)


def _sc_gather_body(tab_ref, idx_ref, out_ref, idx_v, rows_v, sem):
    wid = lax.axis_index("s") * 2 + lax.axis_index("c")
    pltpu.sync_copy(idx_ref.at[wid], idx_v)
    copies = [
        pltpu.async_copy(tab_ref.at[idx_v.at[j]], rows_v.at[j], sem)
        for j in range(_NJ)
    ]
    for c in copies:
        c.wait()
    pltpu.sync_copy(rows_v, out_ref.at[wid])


@functools.cache
def _sc_gather():
    return pl.kernel(
        _sc_gather_body,
        out_type=jax.ShapeDtypeStruct((_NW, _NJ, _JCH, _DIM), jnp.float32),
        mesh=plsc.VectorSubcoreMesh(core_axis_name="c", subcore_axis_name="s"),
        scratch_types=[
            pltpu.VMEM((_NJ, _JCH), jnp.int32),
            pltpu.VMEM((_NJ, _JCH, _DIM), jnp.float32),
            pltpu.SemaphoreType.DMA,
        ],
        compiler_params=pltpu.CompilerParams(use_tc_tiling_on_sc=False),
    )


def kernel(x, embeddings):
    x_flat = x.reshape(_TOK, _DIM)
    idx, loss_sum = _tc_stage(x_flat, embeddings)
    q = _sc_gather()(embeddings.T, idx.reshape(_NW, _NJ, _JCH))
    quantized = q.reshape(x.shape)
    mean_d = loss_sum[0, 0] / jnp.float32(_TOK * _DIM)
    loss = _BETA * mean_d + mean_d
    return quantized, loss
